# baseline (speedup 1.0000x reference)
import functools

import jax
import jax.numpy as jnp
from jax import lax
from jax.experimental import pallas as pl
from jax.experimental.pallas import tpu as pltpu
from jax.experimental.pallas import tpu_sc as plsc

N = 10000
D = 128
D_OUT = 64
E = 320000

NC = 2
NS = 16
NW = NC * NS
CH = 120
NR = 3
NCHUNK = 87
NG = NCHUNK // NR
EPW = NCHUNK * CH
E_PAD = NW * EPW
N_ACC = 10112
RPT = N_ACC // NS


def _mesh():
    return plsc.VectorSubcoreMesh(
        core_axis_name="c", subcore_axis_name="s", num_cores=NC, num_subcores=NS
    )


def _make_prop(width):
    @functools.partial(
        pl.kernel,
        out_type=jax.ShapeDtypeStruct((NC, N_ACC, width), jnp.float32),
        mesh=_mesh(),
        scratch_types=[
            pltpu.VMEM((NR * CH,), jnp.int32),
            pltpu.VMEM((8, CH), jnp.int32),
            pltpu.VMEM((NR, CH, width), jnp.float32),
            pltpu.VMEM_SHARED((N_ACC, width), jnp.float32),
            pltpu.SemaphoreType.DMA,
            pltpu.SemaphoreType.DMA,
            pltpu.SemaphoreType.DMA,
        ],
    )
    def prop(table, gidx, sidx, zeros, out, gring, sring, stage, acc,
             sem0, sem1, sem2):
        sems = (sem0, sem1, sem2)
        c = lax.axis_index("c")
        s = lax.axis_index("s")
        w = c * NS + s
        rb = s * RPT
        pltpu.sync_copy(zeros, acc.at[pl.ds(rb, RPT)])
        plsc.subcore_barrier()

        def body(i, carry):
            goff = (w * NCHUNK + i * NR) * CH
            pltpu.sync_copy(gidx.at[pl.ds(goff, NR * CH)], gring)
            pltpu.sync_copy(sidx.at[w, pl.ds(i * 8, 8)], sring)
            for p in range(NR):
                pltpu.async_copy(
                    table.at[gring.at[pl.ds(p * CH, CH)]], stage.at[p], sems[p])
            for p in range(NR):
                pltpu.make_async_copy(
                    table.at[gring.at[pl.ds(p * CH, CH)]], stage.at[p],
                    sems[p]).wait()
                pltpu.sync_copy(stage.at[p], acc.at[sring.at[p]], add=True)
            return carry

        lax.fori_loop(0, NCHUNK // NR, body, 0)
        plsc.subcore_barrier()
        pltpu.sync_copy(acc.at[pl.ds(rb, RPT)], out.at[c, pl.ds(rb, RPT)])

    return prop


_GRID = 16
_BR = 632


def _dinv_block(d0, d1):
    deg = d0[:, 0:1] + d1[:, 0:1] + 1.0
    return lax.rsqrt(deg)


def _tc_first(xp, w1, d0, d1):
    def body(x_ref, w_ref, d0_ref, d1_ref, o_ref):
        dinv = _dinv_block(d0_ref[...], d1_ref[...])
        h = lax.dot_general(
            x_ref[...], w_ref[...], (((1,), (1,)), ((), ())),
            preferred_element_type=jnp.float32,
        )
        o_ref[...] = h * dinv

    return pl.pallas_call(
        body,
        grid=(_GRID,),
        in_specs=[
            pl.BlockSpec((_BR, D), lambda i: (i, 0)),
            pl.BlockSpec((D, D), lambda i: (0, 0)),
            pl.BlockSpec((_BR, D), lambda i: (i, 0)),
            pl.BlockSpec((_BR, D), lambda i: (i, 0)),
        ],
        out_specs=pl.BlockSpec((_BR, D), lambda i: (i, 0)),
        out_shape=jax.ShapeDtypeStruct((N, D), jnp.float32),
    )(xp, w1, d0, d1)


def _tc_mid(s0, s1, u1, w2, b1, d0, d1):
    def body(s0_ref, s1_ref, u_ref, w_ref, b_ref, d0_ref, d1_ref, o_ref):
        dinv = _dinv_block(d0_ref[...], d1_ref[...])
        agg = s0_ref[...] + s1_ref[...] + u_ref[...]
        h1 = jnp.maximum(agg * dinv + b_ref[...], 0.0)
        h2 = lax.dot_general(
            h1, w_ref[...], (((1,), (1,)), ((), ())),
            preferred_element_type=jnp.float32,
        )
        o_ref[...] = h2 * dinv

    return pl.pallas_call(
        body,
        grid=(_GRID,),
        in_specs=[
            pl.BlockSpec((_BR, D), lambda i: (i, 0)),
            pl.BlockSpec((_BR, D), lambda i: (i, 0)),
            pl.BlockSpec((_BR, D), lambda i: (i, 0)),
            pl.BlockSpec((D, D), lambda i: (0, 0)),
            pl.BlockSpec((1, D), lambda i: (0, 0)),
            pl.BlockSpec((_BR, D), lambda i: (i, 0)),
            pl.BlockSpec((_BR, D), lambda i: (i, 0)),
        ],
        out_specs=pl.BlockSpec((_BR, D), lambda i: (i, 0)),
        out_shape=jax.ShapeDtypeStruct((N, D), jnp.float32),
    )(s0, s1, u1, w2, b1, d0, d1)


def _tc_last(s0, s1, u2, wc, b2, bc, d0, d1):
    def body(s0_ref, s1_ref, u_ref, w_ref, b2_ref, bc_ref, d0_ref, d1_ref, o_ref):
        dinv = _dinv_block(d0_ref[...], d1_ref[...])
        agg = s0_ref[...] + s1_ref[...] + u_ref[...]
        h2 = agg * dinv + b2_ref[...]
        o = lax.dot_general(
            h2, w_ref[...], (((1,), (1,)), ((), ())),
            preferred_element_type=jnp.float32,
        )
        o_ref[...] = o + bc_ref[...]

    return pl.pallas_call(
        body,
        grid=(_GRID,),
        in_specs=[
            pl.BlockSpec((_BR, D), lambda i: (i, 0)),
            pl.BlockSpec((_BR, D), lambda i: (i, 0)),
            pl.BlockSpec((_BR, D), lambda i: (i, 0)),
            pl.BlockSpec((D_OUT, D), lambda i: (0, 0)),
            pl.BlockSpec((1, D), lambda i: (0, 0)),
            pl.BlockSpec((1, D_OUT), lambda i: (0, 0)),
            pl.BlockSpec((_BR, D), lambda i: (i, 0)),
            pl.BlockSpec((_BR, D), lambda i: (i, 0)),
        ],
        out_specs=pl.BlockSpec((_BR, D_OUT), lambda i: (i, 0)),
        out_shape=jax.ShapeDtypeStruct((N, D_OUT), jnp.float32),
    )(s0, s1, u2, wc, b2, bc, d0, d1)


def kernel(x, edge_index, W1, b1, W2, b2, Wc, bc):
    src = edge_index[0]
    dst = edge_index[1]
    padg = jnp.zeros((E_PAD - E,), dtype=jnp.int32)
    pads = jnp.full((E_PAD - E,), N, dtype=jnp.int32)
    srcp = jnp.concatenate([src, padg])
    dstg = jnp.concatenate([dst, padg])
    d3 = jnp.concatenate([dst, pads]).reshape(NW, NG, NR, CH)
    fill = jnp.full((NW, NG, 8 - NR, CH), N, dtype=jnp.int32)
    dsts = jnp.concatenate([d3, fill], axis=2).reshape(NW, NG * 8, CH)

    ones_tab = jnp.ones((N, D), jnp.float32)
    zeros_t = jnp.zeros((RPT, D), jnp.float32)
    b1r = b1.reshape(1, D)
    b2r = b2.reshape(1, D)
    bcr = bc.reshape(1, D_OUT)

    prop = _make_prop(D)
    deg = prop(ones_tab, dstg, dsts, zeros_t)
    d0, d1 = deg[0], deg[1]

    u1 = _tc_first(x, W1, d0, d1)
    s1 = prop(u1, srcp, dsts, zeros_t)
    u2 = _tc_mid(s1[0], s1[1], u1, W2, b1r, d0, d1)
    s2 = prop(u2, srcp, dsts, zeros_t)
    return _tc_last(s2[0], s2[1], u2, Wc, b2r, bcr, d0, d1)


# speedup vs baseline: 4.2145x; 4.2145x over previous
import functools

import jax
import jax.numpy as jnp
from jax import lax
from jax.experimental import pallas as pl
from jax.experimental.pallas import tpu as pltpu
from jax.experimental.pallas import tpu_sc as plsc

N = 10000
D = 128
D_OUT = 64
E = 320000

NC = 2
NS = 16
NW = NC * NS
CH = 120
NR = 3
NCHUNK = 87
NG = NCHUNK // NR
EPW = NCHUNK * CH
E_PAD = NW * EPW
N_ACC = 10112
RPT = N_ACC // NS


def _mesh():
    return plsc.VectorSubcoreMesh(
        core_axis_name="c", subcore_axis_name="s", num_cores=NC, num_subcores=NS
    )


def _make_prop(width):
    @functools.partial(
        pl.kernel,
        out_type=jax.ShapeDtypeStruct((NC, N_ACC, width), jnp.float32),
        mesh=_mesh(),
        scratch_types=[
            pltpu.VMEM((NR * CH,), jnp.int32),
            pltpu.VMEM((8, CH), jnp.int32),
            pltpu.VMEM((NR, CH, width), jnp.float32),
            pltpu.VMEM_SHARED((N_ACC, width), jnp.float32),
            pltpu.SemaphoreType.DMA,
            pltpu.SemaphoreType.DMA,
            pltpu.SemaphoreType.DMA,
        ],
    )
    def prop(table, gidx, sidx, zeros, out, gring, sring, stage, acc,
             sem0, sem1, sem2):
        sems = (sem0, sem1, sem2)
        c = lax.axis_index("c")
        s = lax.axis_index("s")
        w = c * NS + s
        rb = s * RPT
        pltpu.sync_copy(zeros, acc.at[pl.ds(rb, RPT)])
        plsc.subcore_barrier()

        def body(i, carry):
            goff = (w * NCHUNK + i * NR) * CH
            pltpu.sync_copy(gidx.at[pl.ds(goff, NR * CH)], gring)
            pltpu.sync_copy(sidx.at[w, pl.ds(i * 8, 8)], sring)
            for p in range(NR):
                pltpu.async_copy(
                    table.at[gring.at[pl.ds(p * CH, CH)]], stage.at[p], sems[p])
            for p in range(NR):
                pltpu.make_async_copy(
                    table.at[gring.at[pl.ds(p * CH, CH)]], stage.at[p],
                    sems[p]).wait()
                pltpu.sync_copy(stage.at[p], acc.at[sring.at[p]], add=True)
            return carry

        lax.fori_loop(0, NCHUNK // NR, body, 0)
        plsc.subcore_barrier()
        pltpu.sync_copy(acc.at[pl.ds(rb, RPT)], out.at[c, pl.ds(rb, RPT)])

    return prop


_GRID = 16
_BR = 632


def _dinv_block(d0, d1):
    deg = d0[:, 0:1] + d1[:, 0:1] + 1.0
    return lax.rsqrt(deg)


def _tc_first(xp, w1, d0, d1):
    def body(x_ref, w_ref, d0_ref, d1_ref, o_ref):
        dinv = _dinv_block(d0_ref[...], d1_ref[...])
        h = lax.dot_general(
            x_ref[...], w_ref[...], (((1,), (1,)), ((), ())),
            preferred_element_type=jnp.float32,
        )
        o_ref[...] = h * dinv

    return pl.pallas_call(
        body,
        grid=(_GRID,),
        in_specs=[
            pl.BlockSpec((_BR, D), lambda i: (i, 0)),
            pl.BlockSpec((D, D), lambda i: (0, 0)),
            pl.BlockSpec((_BR, D), lambda i: (i, 0)),
            pl.BlockSpec((_BR, D), lambda i: (i, 0)),
        ],
        out_specs=pl.BlockSpec((_BR, D), lambda i: (i, 0)),
        out_shape=jax.ShapeDtypeStruct((N, D), jnp.float32),
    )(xp, w1, d0, d1)


def _tc_mid(s0, s1, u1, w2, b1, d0, d1):
    def body(s0_ref, s1_ref, u_ref, w_ref, b_ref, d0_ref, d1_ref, o_ref):
        dinv = _dinv_block(d0_ref[...], d1_ref[...])
        agg = s0_ref[...] + s1_ref[...] + u_ref[...]
        h1 = jnp.maximum(agg * dinv + b_ref[...], 0.0)
        h2 = lax.dot_general(
            h1, w_ref[...], (((1,), (1,)), ((), ())),
            preferred_element_type=jnp.float32,
        )
        o_ref[...] = h2 * dinv

    return pl.pallas_call(
        body,
        grid=(_GRID,),
        in_specs=[
            pl.BlockSpec((_BR, D), lambda i: (i, 0)),
            pl.BlockSpec((_BR, D), lambda i: (i, 0)),
            pl.BlockSpec((_BR, D), lambda i: (i, 0)),
            pl.BlockSpec((D, D), lambda i: (0, 0)),
            pl.BlockSpec((1, D), lambda i: (0, 0)),
            pl.BlockSpec((_BR, D), lambda i: (i, 0)),
            pl.BlockSpec((_BR, D), lambda i: (i, 0)),
        ],
        out_specs=pl.BlockSpec((_BR, D), lambda i: (i, 0)),
        out_shape=jax.ShapeDtypeStruct((N, D), jnp.float32),
    )(s0, s1, u1, w2, b1, d0, d1)


def _tc_last(s0, s1, u2, wc, b2, bc, d0, d1):
    def body(s0_ref, s1_ref, u_ref, w_ref, b2_ref, bc_ref, d0_ref, d1_ref, o_ref):
        dinv = _dinv_block(d0_ref[...], d1_ref[...])
        agg = s0_ref[...] + s1_ref[...] + u_ref[...]
        h2 = agg * dinv + b2_ref[...]
        o = lax.dot_general(
            h2, w_ref[...], (((1,), (1,)), ((), ())),
            preferred_element_type=jnp.float32,
        )
        o_ref[...] = o + bc_ref[...]

    return pl.pallas_call(
        body,
        grid=(_GRID,),
        in_specs=[
            pl.BlockSpec((_BR, D), lambda i: (i, 0)),
            pl.BlockSpec((_BR, D), lambda i: (i, 0)),
            pl.BlockSpec((_BR, D), lambda i: (i, 0)),
            pl.BlockSpec((D_OUT, D), lambda i: (0, 0)),
            pl.BlockSpec((1, D), lambda i: (0, 0)),
            pl.BlockSpec((1, D_OUT), lambda i: (0, 0)),
            pl.BlockSpec((_BR, D), lambda i: (i, 0)),
            pl.BlockSpec((_BR, D), lambda i: (i, 0)),
        ],
        out_specs=pl.BlockSpec((_BR, D_OUT), lambda i: (i, 0)),
        out_shape=jax.ShapeDtypeStruct((N, D_OUT), jnp.float32),
    )(s0, s1, u2, wc, b2, bc, d0, d1)


def kernel(x, edge_index, W1, b1, W2, b2, Wc, bc):
    src = edge_index[0]
    dst = edge_index[1]
    ar = jnp.arange(E_PAD - E, dtype=jnp.int32)
    padg = ar % N
    pads = N + ar % (N_ACC - N)
    srcp = jnp.concatenate([src, padg])
    dstg = jnp.concatenate([dst, padg])
    d3 = jnp.concatenate([dst, pads]).reshape(NW, NG, NR, CH)
    fill = jnp.full((NW, NG, 8 - NR, CH), N, dtype=jnp.int32)
    dsts = jnp.concatenate([d3, fill], axis=2).reshape(NW, NG * 8, CH)

    ones_tab = jnp.ones((N, D), jnp.float32)
    zeros_t = jnp.zeros((RPT, D), jnp.float32)
    b1r = b1.reshape(1, D)
    b2r = b2.reshape(1, D)
    bcr = bc.reshape(1, D_OUT)

    prop = _make_prop(D)
    deg = prop(ones_tab, dstg, dsts, zeros_t)
    d0, d1 = deg[0], deg[1]

    u1 = _tc_first(x, W1, d0, d1)
    s1 = prop(u1, srcp, dsts, zeros_t)
    u2 = _tc_mid(s1[0], s1[1], u1, W2, b1r, d0, d1)
    s2 = prop(u2, srcp, dsts, zeros_t)
    return _tc_last(s2[0], s2[1], u2, Wc, b2r, bcr, d0, d1)


# speedup vs baseline: 5.1234x; 1.2157x over previous
import functools

import jax
import jax.numpy as jnp
from jax import lax
from jax.experimental import pallas as pl
from jax.experimental.pallas import tpu as pltpu
from jax.experimental.pallas import tpu_sc as plsc

N = 10000
D = 128
D_OUT = 64
E = 320000

NC = 2
NS = 16
NW = NC * NS
CH = 120
NR = 3
NCHUNK = 87
NG = NCHUNK // NR
EPW = NCHUNK * CH
E_PAD = NW * EPW
N_ACC = 10112
RPT = N_ACC // NS


def _mesh():
    return plsc.VectorSubcoreMesh(
        core_axis_name="c", subcore_axis_name="s", num_cores=NC, num_subcores=NS
    )


def _make_prop(width):
    @functools.partial(
        pl.kernel,
        out_type=jax.ShapeDtypeStruct((NC, N_ACC, width), jnp.float32),
        mesh=_mesh(),
        scratch_types=[
            pltpu.VMEM((NR * CH,), jnp.int32),
            pltpu.VMEM((8, CH), jnp.int32),
            pltpu.VMEM((NR, CH, width), jnp.float32),
            pltpu.VMEM_SHARED((N_ACC, width), jnp.float32),
            pltpu.SemaphoreType.DMA,
            pltpu.SemaphoreType.DMA,
            pltpu.SemaphoreType.DMA,
        ],
    )
    def prop(table, gidx, sidx, zeros, out, gring, sring, stage, acc,
             sem0, sem1, sem2):
        sems = (sem0, sem1, sem2)
        c = lax.axis_index("c")
        s = lax.axis_index("s")
        w = c * NS + s
        rb = s * RPT
        pltpu.sync_copy(zeros, acc.at[pl.ds(rb, RPT)])
        plsc.subcore_barrier()

        def body(i, carry):
            goff = (w * NCHUNK + i * NR) * CH
            pltpu.sync_copy(gidx.at[pl.ds(goff, NR * CH)], gring)
            pltpu.sync_copy(sidx.at[w, pl.ds(i * 8, 8)], sring)
            for p in range(NR):
                pltpu.async_copy(
                    table.at[gring.at[pl.ds(p * CH, CH)]], stage.at[p], sems[p])
            for p in range(NR):
                pltpu.make_async_copy(
                    table.at[gring.at[pl.ds(p * CH, CH)]], stage.at[p],
                    sems[p]).wait()
                pltpu.sync_copy(stage.at[p], acc.at[sring.at[p]], add=True)
            return carry

        lax.fori_loop(0, NCHUNK // NR, body, 0)
        plsc.subcore_barrier()
        pltpu.sync_copy(acc.at[pl.ds(rb, RPT)], out.at[c, pl.ds(rb, RPT)])

    return prop


NCHUNK2 = 82
E_PAD2 = NW * NCHUNK2 * 128


def _make_deg():
    @functools.partial(
        pl.kernel,
        out_type=jax.ShapeDtypeStruct((NC, N_ACC, D), jnp.float32),
        mesh=_mesh(),
        scratch_types=[
            pltpu.VMEM((NCHUNK2, 128), jnp.int32),
            pltpu.VMEM((128, D), jnp.float32),
            pltpu.VMEM_SHARED((N_ACC, D), jnp.float32),
        ],
    )
    def deg(ones, sidx, zeros, out, sidx_v, ones_v, acc):
        c = lax.axis_index("c")
        s = lax.axis_index("s")
        w = c * NS + s
        pltpu.sync_copy(sidx.at[w], sidx_v)
        pltpu.sync_copy(ones, ones_v)
        rb = s * RPT
        pltpu.sync_copy(zeros, acc.at[pl.ds(rb, RPT)])
        plsc.subcore_barrier()

        def body(j, carry):
            pltpu.sync_copy(ones_v, acc.at[sidx_v.at[j]], add=True)
            return carry

        lax.fori_loop(0, NCHUNK2, body, 0)
        plsc.subcore_barrier()
        pltpu.sync_copy(acc.at[pl.ds(rb, RPT)], out.at[c, pl.ds(rb, RPT)])

    return deg


_GRID = 16
_BR = 632


def _dinv_block(d0, d1):
    deg = d0[:, 0:1] + d1[:, 0:1] + 1.0
    return lax.rsqrt(deg)


def _tc_first(xp, w1, d0, d1):
    def body(x_ref, w_ref, d0_ref, d1_ref, o_ref):
        dinv = _dinv_block(d0_ref[...], d1_ref[...])
        h = lax.dot_general(
            x_ref[...], w_ref[...], (((1,), (1,)), ((), ())),
            preferred_element_type=jnp.float32,
        )
        o_ref[...] = h * dinv

    return pl.pallas_call(
        body,
        grid=(_GRID,),
        in_specs=[
            pl.BlockSpec((_BR, D), lambda i: (i, 0)),
            pl.BlockSpec((D, D), lambda i: (0, 0)),
            pl.BlockSpec((_BR, D), lambda i: (i, 0)),
            pl.BlockSpec((_BR, D), lambda i: (i, 0)),
        ],
        out_specs=pl.BlockSpec((_BR, D), lambda i: (i, 0)),
        out_shape=jax.ShapeDtypeStruct((N, D), jnp.float32),
    )(xp, w1, d0, d1)


def _tc_mid(s0, s1, u1, w2, b1, d0, d1):
    def body(s0_ref, s1_ref, u_ref, w_ref, b_ref, d0_ref, d1_ref, o_ref):
        dinv = _dinv_block(d0_ref[...], d1_ref[...])
        agg = s0_ref[...] + s1_ref[...] + u_ref[...]
        h1 = jnp.maximum(agg * dinv + b_ref[...], 0.0)
        h2 = lax.dot_general(
            h1, w_ref[...], (((1,), (1,)), ((), ())),
            preferred_element_type=jnp.float32,
        )
        o_ref[...] = h2 * dinv

    return pl.pallas_call(
        body,
        grid=(_GRID,),
        in_specs=[
            pl.BlockSpec((_BR, D), lambda i: (i, 0)),
            pl.BlockSpec((_BR, D), lambda i: (i, 0)),
            pl.BlockSpec((_BR, D), lambda i: (i, 0)),
            pl.BlockSpec((D, D), lambda i: (0, 0)),
            pl.BlockSpec((1, D), lambda i: (0, 0)),
            pl.BlockSpec((_BR, D), lambda i: (i, 0)),
            pl.BlockSpec((_BR, D), lambda i: (i, 0)),
        ],
        out_specs=pl.BlockSpec((_BR, D), lambda i: (i, 0)),
        out_shape=jax.ShapeDtypeStruct((N, D), jnp.float32),
    )(s0, s1, u1, w2, b1, d0, d1)


def _tc_last(s0, s1, u2, wc, b2, bc, d0, d1):
    def body(s0_ref, s1_ref, u_ref, w_ref, b2_ref, bc_ref, d0_ref, d1_ref, o_ref):
        dinv = _dinv_block(d0_ref[...], d1_ref[...])
        agg = s0_ref[...] + s1_ref[...] + u_ref[...]
        h2 = agg * dinv + b2_ref[...]
        o = lax.dot_general(
            h2, w_ref[...], (((1,), (1,)), ((), ())),
            preferred_element_type=jnp.float32,
        )
        o_ref[...] = o + bc_ref[...]

    return pl.pallas_call(
        body,
        grid=(_GRID,),
        in_specs=[
            pl.BlockSpec((_BR, D), lambda i: (i, 0)),
            pl.BlockSpec((_BR, D), lambda i: (i, 0)),
            pl.BlockSpec((_BR, D), lambda i: (i, 0)),
            pl.BlockSpec((D_OUT, D), lambda i: (0, 0)),
            pl.BlockSpec((1, D), lambda i: (0, 0)),
            pl.BlockSpec((1, D_OUT), lambda i: (0, 0)),
            pl.BlockSpec((_BR, D), lambda i: (i, 0)),
            pl.BlockSpec((_BR, D), lambda i: (i, 0)),
        ],
        out_specs=pl.BlockSpec((_BR, D_OUT), lambda i: (i, 0)),
        out_shape=jax.ShapeDtypeStruct((N, D_OUT), jnp.float32),
    )(s0, s1, u2, wc, b2, bc, d0, d1)


def kernel(x, edge_index, W1, b1, W2, b2, Wc, bc):
    src = edge_index[0]
    dst = edge_index[1]
    ar = jnp.arange(E_PAD - E, dtype=jnp.int32)
    padg = ar % N
    pads = N + ar % (N_ACC - N)
    srcp = jnp.concatenate([src, padg])
    d3 = jnp.concatenate([dst, pads]).reshape(NW, NG, NR, CH)
    fill = jnp.full((NW, NG, 8 - NR, CH), N, dtype=jnp.int32)
    dsts = jnp.concatenate([d3, fill], axis=2).reshape(NW, NG * 8, CH)

    ar2 = jnp.arange(E_PAD2 - E, dtype=jnp.int32)
    pads2 = N + ar2 % (N_ACC - N)
    dst2 = jnp.concatenate([dst, pads2]).reshape(NW, NCHUNK2, 128)

    ones_rows = jnp.ones((128, D), jnp.float32)
    zeros_t = jnp.zeros((RPT, D), jnp.float32)
    b1r = b1.reshape(1, D)
    b2r = b2.reshape(1, D)
    bcr = bc.reshape(1, D_OUT)

    prop = _make_prop(D)
    deg = _make_deg()(ones_rows, dst2, zeros_t)
    d0, d1 = deg[0], deg[1]

    u1 = _tc_first(x, W1, d0, d1)
    s1 = prop(u1, srcp, dsts, zeros_t)
    u2 = _tc_mid(s1[0], s1[1], u1, W2, b1r, d0, d1)
    s2 = prop(u2, srcp, dsts, zeros_t)
    return _tc_last(s2[0], s2[1], u2, Wc, b2r, bcr, d0, d1)


# speedup vs baseline: 5.7687x; 1.1259x over previous
import functools

import jax
import jax.numpy as jnp
from jax import lax
from jax.experimental import pallas as pl
from jax.experimental.pallas import tpu as pltpu
from jax.experimental.pallas import tpu_sc as plsc

N = 10000
D = 128
D_OUT = 64
E = 320000

NC = 2
NS = 16
NW = NC * NS
CH = 120
NR = 3
NCHUNK = 87
NG = NCHUNK // NR
EPW = NCHUNK * CH
E_PAD = NW * EPW
N_ACC = 10112
RPT = N_ACC // NS


def _mesh():
    return plsc.VectorSubcoreMesh(
        core_axis_name="c", subcore_axis_name="s", num_cores=NC, num_subcores=NS
    )


def _make_prop(width):
    NB = NCHUNK // NR

    @functools.partial(
        pl.kernel,
        out_type=jax.ShapeDtypeStruct((NC, N_ACC, width), jnp.float32),
        mesh=_mesh(),
        scratch_types=[
            pltpu.VMEM((2, 8, CH), jnp.int32),
            pltpu.VMEM((2, 8, CH), jnp.int32),
            pltpu.VMEM((NR, CH, width), jnp.float32),
            pltpu.VMEM_SHARED((N_ACC, width), jnp.float32),
            pltpu.SemaphoreType.DMA,
            pltpu.SemaphoreType.DMA,
            pltpu.SemaphoreType.DMA,
            pltpu.SemaphoreType.DMA,
            pltpu.SemaphoreType.DMA,
        ],
    )
    def prop(table, gidx, sidx, zeros, out, gring, sring, stage, acc,
             sem0, sem1, sem2, semg, semsx):
        sems = (sem0, sem1, sem2)
        c = lax.axis_index("c")
        s = lax.axis_index("s")
        w = c * NS + s
        rb = s * RPT
        pltpu.async_copy(gidx.at[w, pl.ds(0, 8)], gring.at[0], semg)
        pltpu.async_copy(sidx.at[w, pl.ds(0, 8)], sring.at[0], semsx)
        pltpu.sync_copy(zeros, acc.at[pl.ds(rb, RPT)])
        plsc.subcore_barrier()

        def body(i, carry):
            par = lax.rem(i, 2)
            nxt = lax.rem(i + 1, 2)
            pltpu.make_async_copy(
                gidx.at[w, pl.ds(i * 8, 8)], gring.at[par], semg).wait()
            pltpu.make_async_copy(
                sidx.at[w, pl.ds(i * 8, 8)], sring.at[par], semsx).wait()
            pltpu.async_copy(gidx.at[w, pl.ds((i + 1) * 8, 8)], gring.at[nxt], semg)
            pltpu.async_copy(sidx.at[w, pl.ds((i + 1) * 8, 8)], sring.at[nxt], semsx)
            for p in range(NR):
                pltpu.async_copy(
                    table.at[gring.at[par, p]], stage.at[p], sems[p])
            for p in range(NR):
                pltpu.make_async_copy(
                    table.at[gring.at[par, p]], stage.at[p], sems[p]).wait()
                pltpu.sync_copy(stage.at[p], acc.at[sring.at[par, p]], add=True)
            return carry

        lax.fori_loop(0, NB, body, 0)
        pltpu.make_async_copy(
            gidx.at[w, pl.ds(NB * 8, 8)], gring.at[NB % 2], semg).wait()
        pltpu.make_async_copy(
            sidx.at[w, pl.ds(NB * 8, 8)], sring.at[NB % 2], semsx).wait()
        plsc.subcore_barrier()
        pltpu.sync_copy(acc.at[pl.ds(rb, RPT)], out.at[c, pl.ds(rb, RPT)])

    return prop


NCHUNK2 = 82
E_PAD2 = NW * NCHUNK2 * 128


def _make_deg():
    @functools.partial(
        pl.kernel,
        out_type=jax.ShapeDtypeStruct((NC, N_ACC, D), jnp.float32),
        mesh=_mesh(),
        scratch_types=[
            pltpu.VMEM((NCHUNK2, 128), jnp.int32),
            pltpu.VMEM((128, D), jnp.float32),
            pltpu.VMEM_SHARED((N_ACC, D), jnp.float32),
        ],
    )
    def deg(ones, sidx, zeros, out, sidx_v, ones_v, acc):
        c = lax.axis_index("c")
        s = lax.axis_index("s")
        w = c * NS + s
        pltpu.sync_copy(sidx.at[w], sidx_v)
        pltpu.sync_copy(ones, ones_v)
        rb = s * RPT
        pltpu.sync_copy(zeros, acc.at[pl.ds(rb, RPT)])
        plsc.subcore_barrier()

        def body(j, carry):
            pltpu.sync_copy(ones_v, acc.at[sidx_v.at[j]], add=True)
            return carry

        lax.fori_loop(0, NCHUNK2, body, 0)
        plsc.subcore_barrier()
        pltpu.sync_copy(acc.at[pl.ds(rb, RPT)], out.at[c, pl.ds(rb, RPT)])

    return deg


_GRID = 16
_BR = 632


def _dinv_block(d0, d1):
    deg = d0[:, 0:1] + d1[:, 0:1] + 1.0
    return lax.rsqrt(deg)


def _tc_first(xp, w1, d0, d1):
    def body(x_ref, w_ref, d0_ref, d1_ref, o_ref):
        dinv = _dinv_block(d0_ref[...], d1_ref[...])
        h = lax.dot_general(
            x_ref[...], w_ref[...], (((1,), (1,)), ((), ())),
            preferred_element_type=jnp.float32,
        )
        o_ref[...] = h * dinv

    return pl.pallas_call(
        body,
        grid=(_GRID,),
        in_specs=[
            pl.BlockSpec((_BR, D), lambda i: (i, 0)),
            pl.BlockSpec((D, D), lambda i: (0, 0)),
            pl.BlockSpec((_BR, D), lambda i: (i, 0)),
            pl.BlockSpec((_BR, D), lambda i: (i, 0)),
        ],
        out_specs=pl.BlockSpec((_BR, D), lambda i: (i, 0)),
        out_shape=jax.ShapeDtypeStruct((N, D), jnp.float32),
    )(xp, w1, d0, d1)


def _tc_mid(s0, s1, u1, w2, b1, d0, d1):
    def body(s0_ref, s1_ref, u_ref, w_ref, b_ref, d0_ref, d1_ref, o_ref):
        dinv = _dinv_block(d0_ref[...], d1_ref[...])
        agg = s0_ref[...] + s1_ref[...] + u_ref[...]
        h1 = jnp.maximum(agg * dinv + b_ref[...], 0.0)
        h2 = lax.dot_general(
            h1, w_ref[...], (((1,), (1,)), ((), ())),
            preferred_element_type=jnp.float32,
        )
        o_ref[...] = h2 * dinv

    return pl.pallas_call(
        body,
        grid=(_GRID,),
        in_specs=[
            pl.BlockSpec((_BR, D), lambda i: (i, 0)),
            pl.BlockSpec((_BR, D), lambda i: (i, 0)),
            pl.BlockSpec((_BR, D), lambda i: (i, 0)),
            pl.BlockSpec((D, D), lambda i: (0, 0)),
            pl.BlockSpec((1, D), lambda i: (0, 0)),
            pl.BlockSpec((_BR, D), lambda i: (i, 0)),
            pl.BlockSpec((_BR, D), lambda i: (i, 0)),
        ],
        out_specs=pl.BlockSpec((_BR, D), lambda i: (i, 0)),
        out_shape=jax.ShapeDtypeStruct((N, D), jnp.float32),
    )(s0, s1, u1, w2, b1, d0, d1)


def _tc_last(s0, s1, u2, wc, b2, bc, d0, d1):
    def body(s0_ref, s1_ref, u_ref, w_ref, b2_ref, bc_ref, d0_ref, d1_ref, o_ref):
        dinv = _dinv_block(d0_ref[...], d1_ref[...])
        agg = s0_ref[...] + s1_ref[...] + u_ref[...]
        h2 = agg * dinv + b2_ref[...]
        o = lax.dot_general(
            h2, w_ref[...], (((1,), (1,)), ((), ())),
            preferred_element_type=jnp.float32,
        )
        o_ref[...] = o + bc_ref[...]

    return pl.pallas_call(
        body,
        grid=(_GRID,),
        in_specs=[
            pl.BlockSpec((_BR, D), lambda i: (i, 0)),
            pl.BlockSpec((_BR, D), lambda i: (i, 0)),
            pl.BlockSpec((_BR, D), lambda i: (i, 0)),
            pl.BlockSpec((D_OUT, D), lambda i: (0, 0)),
            pl.BlockSpec((1, D), lambda i: (0, 0)),
            pl.BlockSpec((1, D_OUT), lambda i: (0, 0)),
            pl.BlockSpec((_BR, D), lambda i: (i, 0)),
            pl.BlockSpec((_BR, D), lambda i: (i, 0)),
        ],
        out_specs=pl.BlockSpec((_BR, D_OUT), lambda i: (i, 0)),
        out_shape=jax.ShapeDtypeStruct((N, D_OUT), jnp.float32),
    )(s0, s1, u2, wc, b2, bc, d0, d1)


def kernel(x, edge_index, W1, b1, W2, b2, Wc, bc):
    src = edge_index[0]
    dst = edge_index[1]
    ar = jnp.arange(E_PAD - E, dtype=jnp.int32)
    padg = ar % N
    pads = N + ar % (N_ACC - N)
    fill = jnp.full((NW, NG, 8 - NR, CH), N, dtype=jnp.int32)
    extra = jnp.zeros((NW, 8, CH), dtype=jnp.int32)
    g3 = jnp.concatenate([src, padg]).reshape(NW, NG, NR, CH)
    srcp = jnp.concatenate(
        [jnp.concatenate([g3, fill], axis=2).reshape(NW, NG * 8, CH), extra],
        axis=1)
    d3 = jnp.concatenate([dst, pads]).reshape(NW, NG, NR, CH)
    dsts = jnp.concatenate(
        [jnp.concatenate([d3, fill], axis=2).reshape(NW, NG * 8, CH), extra],
        axis=1)

    ar2 = jnp.arange(E_PAD2 - E, dtype=jnp.int32)
    pads2 = N + ar2 % (N_ACC - N)
    dst2 = jnp.concatenate([dst, pads2]).reshape(NW, NCHUNK2, 128)

    ones_rows = jnp.ones((128, D), jnp.float32)
    zeros_t = jnp.zeros((RPT, D), jnp.float32)
    b1r = b1.reshape(1, D)
    b2r = b2.reshape(1, D)
    bcr = bc.reshape(1, D_OUT)

    prop = _make_prop(D)
    deg = _make_deg()(ones_rows, dst2, zeros_t)
    d0, d1 = deg[0], deg[1]

    u1 = _tc_first(x, W1, d0, d1)
    s1 = prop(u1, srcp, dsts, zeros_t)
    u2 = _tc_mid(s1[0], s1[1], u1, W2, b1r, d0, d1)
    s2 = prop(u2, srcp, dsts, zeros_t)
    return _tc_last(s2[0], s2[1], u2, Wc, b2r, bcr, d0, d1)
